# baseline (speedup 1.0000x reference)
import functools

import numpy as np
import jax
import jax.numpy as jnp
from jax import lax
from jax.experimental import pallas as pl
from jax.experimental.pallas import tpu as pltpu
from jax.experimental.pallas import tpu_sc as plsc

_B = 4096
_F = 13
_D0 = 64
_D1 = 16
_V = 1300000
_FIELD_DIM = 100000

_NC, _NS = 2, 16
_NW = _NC * _NS
_BW = _B // _NW
_CB = 4
_GN = _CB * _F
_GP = 56
_NCH = _BW // _CB

_OFF = np.arange(_F, dtype=np.int32) * _FIELD_DIM


def _make_sc_bag():
    mesh = plsc.VectorSubcoreMesh(core_axis_name="c", subcore_axis_name="s")

    @functools.partial(
        pl.kernel,
        mesh=mesh,
        compiler_params=pltpu.CompilerParams(use_tc_tiling_on_sc=False),
        scratch_types=[
            pltpu.VMEM((_GP,), jnp.int32),
            pltpu.VMEM((_GP,), jnp.int32),
            pltpu.VMEM((_GP,), jnp.int32),
            pltpu.VMEM((_GP,), jnp.int32),
            pltpu.VMEM((_GP,), jnp.int32),
            pltpu.VMEM((_GP,), jnp.int32),
            pltpu.VMEM((_GP,), jnp.int32),
            pltpu.VMEM((_GP,), jnp.int32),
            pltpu.VMEM((_GP, 8 * _D0), jnp.float32),
            pltpu.VMEM((_GP, 8 * _D0), jnp.float32),
            pltpu.VMEM((_GP, 8 * _D1), jnp.float32),
            pltpu.VMEM((_GP, 8 * _D1), jnp.float32),
            pltpu.VMEM((_CB, _D0), jnp.float32),
            pltpu.VMEM((_CB, _D1), jnp.float32),
            pltpu.SemaphoreType.DMA,
            pltpu.SemaphoreType.DMA,
            pltpu.SemaphoreType.DMA,
            pltpu.SemaphoreType.DMA,
        ],
        out_type=(
            jax.ShapeDtypeStruct((_B, _D0), jnp.float32),
            jax.ShapeDtypeStruct((_B, _D1), jnp.float32),
        ),
    )
    def bag(blk0_hbm, blk1_hbm, sub0_hbm, sub1_hbm, t0_hbm, t1_hbm,
            out0_hbm, s1_hbm,
            blk0a, blk0b, blk1a, blk1b, sub0a, sub0b, sub1a, sub1b,
            rows0a, rows0b, rows1a, rows1b,
            acc0_v, acc1_v, sem0a, sem1a, sem0b, sem1b):
        wid = lax.axis_index("s") * _NC + lax.axis_index("c")
        cbase = wid * _NCH
        bufs = (
            (blk0a, blk1a, sub0a, sub1a, rows0a, rows1a, sem0a, sem1a),
            (blk0b, blk1b, sub0b, sub1b, rows0b, rows1b, sem0b, sem1b),
        )

        def fetch(c, buf):
            blk0_v, blk1_v, sub0_v, sub1_v, r0_v, r1_v, s0, s1 = bufs[buf]
            ib = pl.multiple_of((cbase + c) * _GP, 8)
            pltpu.sync_copy(blk0_hbm.at[pl.ds(ib, _GP)], blk0_v)
            pltpu.sync_copy(blk1_hbm.at[pl.ds(ib, _GP)], blk1_v)
            pltpu.sync_copy(sub0_hbm.at[pl.ds(ib, _GP)], sub0_v)
            pltpu.sync_copy(sub1_hbm.at[pl.ds(ib, _GP)], sub1_v)
            pltpu.async_copy(t0_hbm.at[blk0_v], r0_v, s0)
            pltpu.async_copy(t1_hbm.at[blk1_v], r1_v, s1)

        def drain(buf):
            _, _, _, _, r0_v, r1_v, s0, s1 = bufs[buf]
            pltpu.make_async_copy(t0_hbm.at[pl.ds(0, _GP)], r0_v, s0).wait()
            pltpu.make_async_copy(t1_hbm.at[pl.ds(0, _GP)], r1_v, s1).wait()

        def compute(c, buf):
            _, _, sub0_v, sub1_v, r0_v, r1_v, _, _ = bufs[buf]
            rb = (cbase + c) * _CB

            def row_body(r, carry2):
                rf = r * _F
                s0vec = sub0_v[pl.ds(rf, 16)] * _D0
                s1vec = sub1_v[pl.ds(rf, 16)] * _D1
                for k in range(_D0 // 16):
                    acc = r0_v[rf, pl.ds(s0vec[0] + k * 16, 16)]
                    for j in range(1, _F):
                        acc = acc + r0_v[rf + j, pl.ds(s0vec[j] + k * 16, 16)]
                    acc0_v[r, pl.ds(k * 16, 16)] = acc
                a1 = r1_v[rf, pl.ds(s1vec[0], 16)]
                for j in range(1, _F):
                    a1 = a1 + r1_v[rf + j, pl.ds(s1vec[j], 16)]
                acc1_v[r, :] = a1
                return carry2

            lax.fori_loop(0, _CB, row_body, 0)
            pltpu.sync_copy(acc0_v, out0_hbm.at[pl.ds(rb, _CB)])
            pltpu.sync_copy(acc1_v, s1_hbm.at[pl.ds(rb, _CB)])

        fetch(0, 0)

        def pair_body(p, carry):
            c0 = p * 2
            fetch(c0 + 1, 1)
            drain(0)
            compute(c0, 0)
            fetch(lax.min(c0 + 2, _NCH - 1), 0)
            drain(1)
            compute(c0 + 1, 1)
            return carry

        lax.fori_loop(0, _NCH // 2, pair_body, 0)
        drain(0)

    return bag


_SC_BAG = _make_sc_bag()


def _combine_body(out0_ref, s1_ref, pw_ref, pb_ref, o_ref):
    proj = lax.dot_general(
        s1_ref[...], pw_ref[...],
        dimension_numbers=(((1,), (1,)), ((), ())),
        preferred_element_type=jnp.float32,
    )
    o_ref[...] = out0_ref[...] + proj + np.float32(_F) * pb_ref[...]


_COMBINE = pl.pallas_call(
    _combine_body,
    out_shape=jax.ShapeDtypeStruct((_B, _D0), jnp.float32),
)


def _chunk_pad(a):
    a = a.reshape(_B // _CB, _CB * _F)
    pad = jnp.zeros((_B // _CB, _GP - _CB * _F), dtype=a.dtype)
    return jnp.concatenate([a, pad], axis=1).reshape(-1)


@jax.jit
def kernel(x, table0, table1, proj_w, proj_b):
    x = x.astype(jnp.int32)
    off = jnp.asarray(_OFF)
    idx0 = x[:, :_F] + off[None, :]
    idx1 = x[:, _F:] + off[None, :]
    t0 = table0.astype(jnp.float32).reshape(_V // 8, 8 * _D0)
    t1 = table1.astype(jnp.float32).reshape(_V // 8, 8 * _D1)
    out0, s1 = _SC_BAG(_chunk_pad(idx0 >> 3), _chunk_pad(idx1 >> 3),
                       _chunk_pad(idx0 & 7), _chunk_pad(idx1 & 7), t0, t1)
    return _COMBINE(out0, s1, proj_w.astype(jnp.float32),
                    proj_b.reshape(1, _D0).astype(jnp.float32))


# speedup vs baseline: 2.5378x; 2.5378x over previous
import functools

import numpy as np
import jax
import jax.numpy as jnp
from jax import lax
from jax.experimental import pallas as pl
from jax.experimental.pallas import tpu as pltpu
from jax.experimental.pallas import tpu_sc as plsc

_B = 4096
_F = 13
_D0 = 64
_D1 = 16
_V = 1300000
_FIELD_DIM = 100000

_NC, _NS = 2, 16
_NW = _NC * _NS
_BW = _B // _NW
_CB = 8
_GN = _CB * _F
_GNP = 112
_NCH = _BW // _CB

_OFF = np.arange(_F, dtype=np.int32) * _FIELD_DIM


def _make_sc_bag():
    mesh = plsc.VectorSubcoreMesh(core_axis_name="c", subcore_axis_name="s")

    @functools.partial(
        pl.kernel,
        mesh=mesh,
        scratch_types=[
            pltpu.VMEM((2, _GNP), jnp.int32),
            pltpu.VMEM((2, _GNP), jnp.int32),
            pltpu.VMEM((2, _GNP), jnp.int32),
            pltpu.VMEM((2, _GNP), jnp.int32),
            pltpu.VMEM((2, _GN, _D0), jnp.float32),
            pltpu.VMEM((2, _GN, _D1), jnp.float32),
            pltpu.VMEM((_CB, _D0), jnp.float32),
            pltpu.VMEM((_CB, _D1), jnp.float32),
            pltpu.SemaphoreType.DMA,
            pltpu.SemaphoreType.DMA,
            pltpu.SemaphoreType.DMA,
            pltpu.SemaphoreType.DMA,
        ],
        out_type=(
            jax.ShapeDtypeStruct((_B, _D0), jnp.float32),
            jax.ShapeDtypeStruct((_B, _D1), jnp.float32),
        ),
    )
    def bag(blk0_hbm, blk1_hbm, sub0_hbm, sub1_hbm, t0_hbm, t1_hbm,
            out0_hbm, s1_hbm,
            blk0_v, blk1_v, sub0_v, sub1_v, rows0_v, rows1_v,
            acc0_v, acc1_v, sem0a, sem1a, sem0b, sem1b):
        wid = lax.axis_index("s") * _NC + lax.axis_index("c")
        base = wid * _BW
        sems = ((sem0a, sem1a), (sem0b, sem1b))

        def fetch(c, buf):
            rb = base + c * _CB
            ib = pl.multiple_of(rb * _F, 8)
            s0, s1 = sems[buf]
            pltpu.sync_copy(blk0_hbm.at[pl.ds(ib, _GN)],
                            blk0_v.at[buf, pl.ds(0, _GN)])
            pltpu.sync_copy(blk1_hbm.at[pl.ds(ib, _GN)],
                            blk1_v.at[buf, pl.ds(0, _GN)])
            pltpu.sync_copy(sub0_hbm.at[pl.ds(ib, _GN)],
                            sub0_v.at[buf, pl.ds(0, _GN)])
            pltpu.sync_copy(sub1_hbm.at[pl.ds(ib, _GN)],
                            sub1_v.at[buf, pl.ds(0, _GN)])
            for gv in range(_GNP // 16):
                b0 = blk0_v[buf, pl.ds(16 * gv, 16)]
                c0 = sub0_v[buf, pl.ds(16 * gv, 16)]
                b1 = blk1_v[buf, pl.ds(16 * gv, 16)]
                c1 = sub1_v[buf, pl.ds(16 * gv, 16)]
                for lane in range(16):
                    g = 16 * gv + lane
                    if g >= _GN:
                        break
                    pltpu.async_copy(t0_hbm.at[b0[lane], c0[lane]],
                                     rows0_v.at[buf, g], s0)
                    pltpu.async_copy(t1_hbm.at[b1[lane], c1[lane]],
                                     rows1_v.at[buf, g], s1)

        def drain(buf):
            s0, s1 = sems[buf]
            pltpu.make_async_copy(out0_hbm.at[pl.ds(0, _GN)],
                                  rows0_v.at[buf], s0).wait()
            pltpu.make_async_copy(s1_hbm.at[pl.ds(0, _GN)],
                                  rows1_v.at[buf], s1).wait()

        def compute(c, buf):
            rb = base + c * _CB

            def row_body(r, carry2):
                rf = r * _F
                for k in range(_D0 // 16):
                    col = pl.ds(k * 16, 16)
                    acc = rows0_v[buf, rf, col]
                    for j in range(1, _F):
                        acc = acc + rows0_v[buf, rf + j, col]
                    acc0_v[r, col] = acc
                a1 = rows1_v[buf, rf, :]
                for j in range(1, _F):
                    a1 = a1 + rows1_v[buf, rf + j, :]
                acc1_v[r, :] = a1
                return carry2

            lax.fori_loop(0, _CB, row_body, 0)
            pltpu.sync_copy(acc0_v, out0_hbm.at[pl.ds(rb, _CB)])
            pltpu.sync_copy(acc1_v, s1_hbm.at[pl.ds(rb, _CB)])

        fetch(0, 0)

        def pair_body(p, carry):
            c0 = p * 2
            fetch(c0 + 1, 1)
            drain(0)
            compute(c0, 0)
            fetch(lax.min(c0 + 2, _NCH - 1), 0)
            drain(1)
            compute(c0 + 1, 1)
            return carry

        lax.fori_loop(0, _NCH // 2, pair_body, 0)
        drain(0)

    return bag


_SC_BAG = _make_sc_bag()


def _combine_body(out0_ref, s1_ref, pw_ref, pb_ref, o_ref):
    proj = lax.dot_general(
        s1_ref[...], pw_ref[...],
        dimension_numbers=(((1,), (1,)), ((), ())),
        preferred_element_type=jnp.float32,
    )
    o_ref[...] = out0_ref[...] + proj + np.float32(_F) * pb_ref[...]


_COMBINE = pl.pallas_call(
    _combine_body,
    out_shape=jax.ShapeDtypeStruct((_B, _D0), jnp.float32),
)


@jax.jit
def kernel(x, table0, table1, proj_w, proj_b):
    x = x.astype(jnp.int32)
    off = jnp.asarray(_OFF)
    idx0 = (x[:, :_F] + off[None, :]).reshape(-1)
    idx1 = (x[:, _F:] + off[None, :]).reshape(-1)
    t0 = table0.astype(jnp.float32).reshape(_V // 8, 8, _D0)
    t1 = table1.astype(jnp.float32).reshape(_V // 8, 8, _D1)
    out0, s1 = _SC_BAG(idx0 >> 3, idx1 >> 3, idx0 & 7, idx1 & 7, t0, t1)
    return _COMBINE(out0, s1, proj_w.astype(jnp.float32),
                    proj_b.reshape(1, _D0).astype(jnp.float32))
